# baseline (speedup 1.0000x reference)
import functools
import math

import jax
import jax.numpy as jnp
from jax import lax
from jax.experimental import pallas as pl
from jax.experimental.pallas import tpu as pltpu
from jax.experimental.pallas import tpu_sc as plsc

B = 1024
D = 64
K = 100000
TEMP = 0.05
CK = 2048
NSTEPS = (K + CK - 1) // CK
LOG2E = math.log2(math.e)
SCALE = LOG2E / TEMP

SC_CORES = 2
SC_SUBCORES = 16
NW = SC_CORES * SC_SUBCORES
BPW = B // NW


BPC = B // SC_CORES


def _gather_rows(table_hbm, idx_hbm, out_hbm, idx_s, sem):
    cid = lax.axis_index("c")
    base = cid * BPC

    pltpu.sync_copy(idx_hbm.at[pl.ds(base, BPC)], idx_s)

    def body(j, carry):
        idx = idx_s[j]
        pltpu.async_copy(table_hbm.at[pl.ds(idx, 1), :],
                         out_hbm.at[pl.ds(base + j, 1), :], sem)
        return carry

    lax.fori_loop(0, BPC, body, 0)

    def drain(j, carry):
        pltpu.make_async_copy(table_hbm.at[pl.ds(0, 1), :],
                              out_hbm.at[pl.ds(base, 1), :], sem).wait()
        return carry

    lax.fori_loop(0, BPC, drain, 0)


def _sc_gather(table, idx):
    mesh = plsc.ScalarSubcoreMesh(axis_name="c", num_cores=SC_CORES)
    return pl.kernel(
        _gather_rows,
        mesh=mesh,
        out_type=jax.ShapeDtypeStruct((B, D), jnp.float32),
        scratch_types=[
            pltpu.SMEM((BPC,), jnp.int32),
            pltpu.SemaphoreType.DMA,
        ],
    )(table, idx)


def _slice_sum(e):
    s = e[:, 0:128]
    for j in range(1, CK // 128):
        s = s + e[:, j * 128:(j + 1) * 128]
    return s


def _sum_kernel(x_ref, f_ref, s_ref, xn_ref, acc_ref):
    i = pl.program_id(0)

    @pl.when(i == 0)
    def _init():
        x = x_ref[...]
        norm = jnp.sqrt(jnp.sum(x * x, axis=1, keepdims=True))
        xn_ref[...] = (x * (SCALE / jnp.maximum(norm, 1e-12))).astype(jnp.bfloat16)
        acc_ref[...] = jnp.zeros_like(acc_ref)

    logits2 = jax.lax.dot_general(
        xn_ref[...], f_ref[...], (((1,), (1,)), ((), ())),
        preferred_element_type=jnp.float32)

    @pl.when(i < NSTEPS - 1)
    def _acc():
        acc_ref[...] += _slice_sum(jnp.exp2(logits2))

    @pl.when(i == NSTEPS - 1)
    def _fini():
        col = i * CK + jax.lax.broadcasted_iota(jnp.int32, (B, CK), 1)
        e = jnp.where(col < K, jnp.exp2(logits2), 0.0)
        acc = acc_ref[...] + _slice_sum(e)
        s_ref[...] = jnp.sum(acc, axis=1, keepdims=True)


def _combine_kernel(x_ref, g_ref, s_ref, out_ref):
    x = x_ref[...]
    norm = jnp.sqrt(jnp.sum(x * x, axis=1, keepdims=True))
    xn = x * ((1.0 / TEMP) / jnp.maximum(norm, 1e-12))
    tgt = jnp.sum(xn * g_ref[...], axis=1, keepdims=True)
    out_ref[...] = jnp.mean(jnp.log(s_ref[...]) - tgt).reshape(1, 1)


@jax.jit
def _run(inputs, corrected_targets, features):
    ct = corrected_targets.astype(jnp.int32)
    g = _sc_gather(features, ct)
    s = pl.pallas_call(
        _sum_kernel,
        grid=(NSTEPS,),
        in_specs=[
            pl.BlockSpec((B, D), lambda i: (0, 0)),
            pl.BlockSpec((CK, D), lambda i: (i, 0)),
        ],
        out_specs=pl.BlockSpec((B, 1), lambda i: (0, 0)),
        out_shape=jax.ShapeDtypeStruct((B, 1), jnp.float32),
        scratch_shapes=[
            pltpu.VMEM((B, D), jnp.bfloat16),
            pltpu.VMEM((B, 128), jnp.float32),
        ],
    )(inputs, features.astype(jnp.bfloat16))
    out = pl.pallas_call(
        _combine_kernel,
        out_shape=jax.ShapeDtypeStruct((1, 1), jnp.float32),
    )(inputs, g, s)
    return out[0, 0]


def kernel(inputs, targets, corrected_targets, features):
    del targets
    return _run(inputs, corrected_targets, features)


# speedup vs baseline: 1.2030x; 1.2030x over previous
import functools
import math

import jax
import jax.numpy as jnp
from jax import lax
from jax.experimental import pallas as pl
from jax.experimental.pallas import tpu as pltpu
from jax.experimental.pallas import tpu_sc as plsc

B = 1024
D = 64
K = 100000
TEMP = 0.05
CK = 2048
NSTEPS = (K + CK - 1) // CK
LOG2E = math.log2(math.e)
SCALE = LOG2E / TEMP

SC_CORES = 2
SC_SUBCORES = 16
NW = SC_CORES * SC_SUBCORES
BPW = B // NW


BPC = B // SC_CORES


def _gather_rows(table_hbm, idx_hbm, out_hbm, idx_s, sem):
    cid = lax.axis_index("c")
    base = cid * BPC

    pltpu.sync_copy(idx_hbm.at[pl.ds(base, BPC)], idx_s)

    def body(j, carry):
        idx = idx_s[j]
        pltpu.async_copy(table_hbm.at[pl.ds(idx, 1), :],
                         out_hbm.at[pl.ds(base + j, 1), :], sem)
        return carry

    lax.fori_loop(0, BPC, body, 0)

    def drain(j, carry):
        pltpu.make_async_copy(table_hbm.at[pl.ds(0, 1), :],
                              out_hbm.at[pl.ds(base, 1), :], sem).wait()
        return carry

    lax.fori_loop(0, BPC, drain, 0)


def _sc_gather(table, idx):
    mesh = plsc.ScalarSubcoreMesh(axis_name="c", num_cores=SC_CORES)
    return pl.kernel(
        _gather_rows,
        mesh=mesh,
        out_type=jax.ShapeDtypeStruct((B, D), jnp.float32),
        scratch_types=[
            pltpu.SMEM((BPC,), jnp.int32),
            pltpu.SemaphoreType.DMA,
        ],
    )(table, idx)


def _slice_sum(e):
    s = e[:, 0:128]
    for j in range(1, CK // 128):
        s = s + e[:, j * 128:(j + 1) * 128]
    return s


def _sum_kernel(x_ref, f_ref, s_ref, xn_ref, acc_ref):
    i = pl.program_id(0)

    @pl.when(i == 0)
    def _init():
        x = x_ref[...]
        norm = jnp.sqrt(jnp.sum(x * x, axis=1, keepdims=True))
        xn_ref[...] = x * (SCALE / jnp.maximum(norm, 1e-12))
        acc_ref[...] = jnp.zeros_like(acc_ref)

    logits2 = jax.lax.dot_general(
        xn_ref[...], f_ref[...], (((1,), (1,)), ((), ())),
        preferred_element_type=jnp.float32)

    @pl.when(i < NSTEPS - 1)
    def _acc():
        acc_ref[...] += _slice_sum(jnp.exp2(logits2))

    @pl.when(i == NSTEPS - 1)
    def _fini():
        col = i * CK + jax.lax.broadcasted_iota(jnp.int32, (B, CK), 1)
        e = jnp.where(col < K, jnp.exp2(logits2), 0.0)
        acc = acc_ref[...] + _slice_sum(e)
        s_ref[...] = jnp.sum(acc, axis=1, keepdims=True)


def _combine_kernel(x_ref, g_ref, s_ref, out_ref):
    x = x_ref[...]
    norm = jnp.sqrt(jnp.sum(x * x, axis=1, keepdims=True))
    xn = x * ((1.0 / TEMP) / jnp.maximum(norm, 1e-12))
    tgt = jnp.sum(xn * g_ref[...], axis=1, keepdims=True)
    out_ref[...] = jnp.mean(jnp.log(s_ref[...]) - tgt).reshape(1, 1)


@jax.jit
def _run(inputs, corrected_targets, features):
    ct = corrected_targets.astype(jnp.int32)
    g = _sc_gather(features, ct)
    s = pl.pallas_call(
        _sum_kernel,
        grid=(NSTEPS,),
        in_specs=[
            pl.BlockSpec((B, D), lambda i: (0, 0)),
            pl.BlockSpec((CK, D), lambda i: (i, 0)),
        ],
        out_specs=pl.BlockSpec((B, 1), lambda i: (0, 0)),
        out_shape=jax.ShapeDtypeStruct((B, 1), jnp.float32),
        scratch_shapes=[
            pltpu.VMEM((B, D), jnp.float32),
            pltpu.VMEM((B, 128), jnp.float32),
        ],
    )(inputs, features)
    out = pl.pallas_call(
        _combine_kernel,
        out_shape=jax.ShapeDtypeStruct((1, 1), jnp.float32),
    )(inputs, g, s)
    return out[0, 0]


def kernel(inputs, targets, corrected_targets, features):
    del targets
    return _run(inputs, corrected_targets, features)
